# baseline (speedup 1.0000x reference)
import functools

import jax
import jax.numpy as jnp
from jax import lax
from jax.experimental import pallas as pl
from jax.experimental.pallas import tpu as pltpu
from jax.experimental.pallas import tpu_sc as plsc

VOCAB = 100000
D_MODEL = 64
NUM_WORKERS = 32
D_PER_W = D_MODEL // NUM_WORKERS
LANES = 16


BLK = 56


def _lookup_body(ids_hbm, table_hbm, out_hbm, row_v, ids_sh, ids_bufs,
                 out_bufs, isems, osems, rsem):
    n_s, n_b = ids_hbm.shape
    w = lax.axis_index("s") * 2 + lax.axis_index("c")
    blocks = []
    s0 = 0
    while s0 < n_s:
        blocks.append((s0, min(BLK, n_s - s0)))
        s0 += BLK

    for k in range(D_PER_W):
        d = w * D_PER_W + k
        row_cp = pltpu.make_async_copy(table_hbm.at[d], row_v, rsem)
        row_cp.start()

        for s0, sz in blocks:
            plsc.subcore_barrier()
            @pl.when(lax.axis_index("s") == 0)
            def _():
                pltpu.sync_copy(ids_hbm.at[pl.ds(s0, sz)],
                                ids_sh.at[pl.ds(0, sz)])

            plsc.subcore_barrier()

            if s0 == 0:
                row_cp.wait()

            for h in range(2):
                pltpu.async_copy(ids_sh.at[h], ids_bufs[h], isems[h])

            def sloop(i, carry):
                for h in range(2):
                    sl = 2 * i + h
                    pltpu.make_async_copy(ids_sh.at[0], ids_bufs[h],
                                          isems[h]).wait()
                    @pl.when(i > 0)
                    def _():
                        pltpu.make_async_copy(out_bufs[h], out_hbm.at[0, d],
                                              osems[h]).wait()

                    @plsc.parallel_loop(0, n_b, LANES, unroll=16)
                    def inner(j):
                        idx = ids_bufs[h][pl.ds(j, LANES)]
                        out_bufs[h][pl.ds(j, LANES)] = plsc.load_gather(
                            row_v, [idx])

                    @pl.when(i < sz // 2 - 1)
                    def _():
                        pltpu.async_copy(ids_sh.at[sl + 2], ids_bufs[h],
                                         isems[h])

                    pltpu.async_copy(out_bufs[h], out_hbm.at[s0 + sl, d],
                                     osems[h])
                return carry

            lax.fori_loop(0, sz // 2, sloop, 0)
            for h in range(2):
                pltpu.make_async_copy(out_bufs[h], out_hbm.at[0, d],
                                      osems[h]).wait()


def kernel(time_ids, pe_weight):
    b, s = time_ids.shape
    ids_t = time_ids.T
    table_t = pe_weight.T

    mesh = plsc.VectorSubcoreMesh(core_axis_name="c", subcore_axis_name="s")
    run = functools.partial(
        pl.kernel,
        mesh=mesh,
        out_type=jax.ShapeDtypeStruct((s, D_MODEL, b), jnp.float32),
        scratch_types=[
            pltpu.VMEM((VOCAB,), jnp.float32),
            pltpu.VMEM_SHARED((BLK, b), jnp.int32),
            [pltpu.VMEM((b,), jnp.int32) for _ in range(2)],
            [pltpu.VMEM((b,), jnp.float32) for _ in range(2)],
            [pltpu.SemaphoreType.DMA for _ in range(2)],
            [pltpu.SemaphoreType.DMA for _ in range(2)],
            pltpu.SemaphoreType.DMA,
        ],
        compiler_params=pltpu.CompilerParams(needs_layout_passes=False),
    )(_lookup_body)
    out_t = run(ids_t, table_t)
    return out_t.transpose(2, 0, 1)
